# Initial kernel scaffold; baseline (speedup 1.0000x reference)
import functools

import jax
import jax.numpy as jnp
from jax import lax
from jax.experimental import pallas as pl
from jax.experimental.pallas import tpu as pltpu
from jax.experimental.pallas import tpu_sc as plsc

B, S, D = 4096, 200, 64
NTOT = B * S
NC, NS = 2, 16
NW = NC * NS
PER_W = NTOT // NW
IDX_ROW = 128
CHUNK = 512
NSUB = CHUNK // IDX_ROW
NSTEPS = PER_W // CHUNK


def _gather_body(idx_hbm, table_hbm, out_hbm, idx_v, rows_v, sem_idx, sem_g):
    wid = lax.axis_index("s") * NC + lax.axis_index("c")
    idx_base = wid * (PER_W // IDX_ROW)
    out_base = wid * PER_W

    def step(g, _):
        pltpu.sync_copy(idx_hbm.at[pl.ds(idx_base + g * NSUB, NSUB)], idx_v)
        copies = []
        for j in range(NSUB):
            copies.append(pltpu.async_copy(
                table_hbm.at[idx_v.at[j]],
                rows_v.at[pl.ds(j * IDX_ROW, IDX_ROW)],
                sem_g))
        for c in copies:
            c.wait()
        pltpu.sync_copy(rows_v, out_hbm.at[pl.ds(out_base + g * CHUNK, CHUNK)])
        return 0

    lax.fori_loop(0, NSTEPS, step, 0)


@jax.jit
def kernel(recipe_indices, embedding_weight):
    idx2d = recipe_indices.reshape(NTOT // IDX_ROW, IDX_ROW).astype(jnp.int32)
    mesh = plsc.VectorSubcoreMesh(
        core_axis_name="c", subcore_axis_name="s",
        num_cores=NC, num_subcores=NS)
    out = pl.kernel(
        _gather_body,
        out_type=jax.ShapeDtypeStruct((NTOT, D), jnp.float32),
        mesh=mesh,
        scratch_types=[
            pltpu.VMEM((NSUB, IDX_ROW), jnp.int32),
            pltpu.VMEM((CHUNK, D), jnp.float32),
            pltpu.SemaphoreType.DMA,
            pltpu.SemaphoreType.DMA,
        ],
    )(idx2d, embedding_weight)
    return out.reshape(B, S, D)


# speedup vs baseline: 3.9485x; 3.9485x over previous
import jax
import jax.numpy as jnp
from jax import lax
from jax.experimental import pallas as pl
from jax.experimental.pallas import tpu as pltpu
from jax.experimental.pallas import tpu_sc as plsc

B, S, D = 4096, 200, 64
NTOT = B * S
NC, NS = 2, 16
NW = NC * NS
PER_W = NTOT // NW
IDX_ROW = 128
CHUNK = 512
NSUB = CHUNK // IDX_ROW
NSTEPS = PER_W // CHUNK


def _gather_body(idx_hbm, table_hbm, out_hbm, idx_v, rows_v, sem_g):
    wid = lax.axis_index("s") * NC + lax.axis_index("c")
    idx_base = wid * (PER_W // IDX_ROW)
    out_base = wid * PER_W

    def step(g, _):
        pltpu.sync_copy(idx_hbm.at[pl.ds(idx_base + g * NSUB, NSUB)], idx_v)
        copies = []
        for j in range(NSUB):
            copies.append(pltpu.async_copy(
                table_hbm.at[idx_v.at[j]],
                rows_v.at[pl.ds(j * IDX_ROW, IDX_ROW)],
                sem_g))
        for c in copies:
            c.wait()
        pltpu.sync_copy(rows_v, out_hbm.at[pl.ds(out_base + g * CHUNK, CHUNK)])
        return 0

    lax.fori_loop(0, NSTEPS, step, 0)


@jax.jit
def kernel(recipe_indices, embedding_weight):
    idx2d = recipe_indices.reshape(NTOT // IDX_ROW, IDX_ROW).astype(jnp.int32)
    mesh = plsc.VectorSubcoreMesh(
        core_axis_name="c", subcore_axis_name="s",
        num_cores=NC, num_subcores=NS)
    out = pl.kernel(
        _gather_body,
        out_type=jax.ShapeDtypeStruct((NTOT, D), jnp.float32),
        mesh=mesh,
        scratch_types=[
            pltpu.VMEM((NSUB, IDX_ROW), jnp.int32),
            pltpu.VMEM((CHUNK, D), jnp.float32),
            pltpu.SemaphoreType.DMA,
        ],
        compiler_params=pltpu.CompilerParams(use_tc_tiling_on_sc=False),
    )(idx2d, embedding_weight)
    return out.reshape(B, S, D)


# speedup vs baseline: 4.2326x; 1.0720x over previous
import jax
import jax.numpy as jnp
from jax import lax
from jax.experimental import pallas as pl
from jax.experimental.pallas import tpu as pltpu
from jax.experimental.pallas import tpu_sc as plsc

B, S, D = 4096, 200, 64
NTOT = B * S
NC, NS = 2, 16
NW = NC * NS
PER_W = NTOT // NW
IDX_ROW = 128
CHUNK = 512
NSUB = CHUNK // IDX_ROW
NSTEPS = PER_W // CHUNK


def _gather_body(idx_hbm, table_hbm, out_hbm, idx_v, rows_v,
                 sem_idx, sem_g, sem_out):
    wid = lax.axis_index("s") * NC + lax.axis_index("c")
    idx_base = wid * (PER_W // IDX_ROW)
    out_base = wid * PER_W

    def idx_slice(g):
        return idx_hbm.at[pl.ds(idx_base + g * NSUB, NSUB)]

    def out_slice(g):
        return out_hbm.at[pl.ds(out_base + g * CHUNK, CHUNK)]

    pltpu.async_copy(idx_slice(0), idx_v.at[0], sem_idx)
    pltpu.async_copy(idx_slice(1), idx_v.at[1], sem_idx)

    def pair_body(p, _):
        for b in range(2):
            g = 2 * p + b
            pltpu.make_async_copy(idx_slice(g), idx_v.at[b], sem_idx).wait()
            @pl.when(p > 0)
            def _():
                pltpu.make_async_copy(rows_v.at[b], out_slice(g - 2),
                                      sem_out).wait()
            copies = []
            for j in range(NSUB):
                copies.append(pltpu.async_copy(
                    table_hbm.at[idx_v.at[b].at[j]],
                    rows_v.at[b].at[pl.ds(j * IDX_ROW, IDX_ROW)],
                    sem_g))
            for c in copies:
                c.wait()
            gn = lax.min(g + 2, NSTEPS - 1)
            pltpu.async_copy(idx_slice(gn), idx_v.at[b], sem_idx)
            pltpu.async_copy(rows_v.at[b], out_slice(g), sem_out)
        return 0

    lax.fori_loop(0, NSTEPS // 2, pair_body, 0)

    for b in range(2):
        g = NSTEPS - 2 + b
        pltpu.make_async_copy(rows_v.at[b], out_slice(g), sem_out).wait()
        pltpu.make_async_copy(idx_slice(NSTEPS - 1), idx_v.at[b],
                              sem_idx).wait()


@jax.jit
def kernel(recipe_indices, embedding_weight):
    idx2d = recipe_indices.reshape(NTOT // IDX_ROW, IDX_ROW).astype(jnp.int32)
    mesh = plsc.VectorSubcoreMesh(
        core_axis_name="c", subcore_axis_name="s",
        num_cores=NC, num_subcores=NS)
    out = pl.kernel(
        _gather_body,
        out_type=jax.ShapeDtypeStruct((NTOT, D), jnp.float32),
        mesh=mesh,
        scratch_types=[
            pltpu.VMEM((2, NSUB, IDX_ROW), jnp.int32),
            pltpu.VMEM((2, CHUNK, D), jnp.float32),
            pltpu.SemaphoreType.DMA,
            pltpu.SemaphoreType.DMA,
            pltpu.SemaphoreType.DMA,
        ],
        compiler_params=pltpu.CompilerParams(use_tc_tiling_on_sc=False),
    )(idx2d, embedding_weight)
    return out.reshape(B, S, D)


# speedup vs baseline: 4.2598x; 1.0064x over previous
import jax
import jax.numpy as jnp
from jax import lax
from jax.experimental import pallas as pl
from jax.experimental.pallas import tpu as pltpu
from jax.experimental.pallas import tpu_sc as plsc

B, S, D = 4096, 200, 64
NC, NS = 2, 16
NW = NC * NS
BPW = B // NW
NBUF = 4


def _gather_body(idxT_hbm, table_hbm, out_hbm, idx_v, rows_v,
                 sem_idx, sem_g0, sem_g1, sem_g2, sem_g3,
                 sem_w0, sem_w1, sem_w2, sem_w3):
    sem_g = [sem_g0, sem_g1, sem_g2, sem_g3]
    sem_w = [sem_w0, sem_w1, sem_w2, sem_w3]
    wid = lax.axis_index("s") * NC + lax.axis_index("c")
    b0 = wid * BPW

    pltpu.sync_copy(idxT_hbm.at[:, pl.ds(b0, BPW)], idx_v)

    def gather(s, buf):
        pltpu.async_copy(table_hbm.at[idx_v.at[s]],
                         rows_v.at[buf], sem_g[buf])

    def gather_wait(s, buf):
        pltpu.make_async_copy(table_hbm.at[idx_v.at[s]],
                              rows_v.at[buf], sem_g[buf]).wait()

    def writeback(s, buf):
        pltpu.async_copy(rows_v.at[buf],
                         out_hbm.at[pl.ds(b0, BPW), s], sem_w[buf])

    def writeback_wait(s, buf):
        pltpu.make_async_copy(rows_v.at[buf],
                              out_hbm.at[pl.ds(b0, BPW), s], sem_w[buf]).wait()

    for s in range(NBUF - 1):
        gather(s, s)

    def quad(p, _):
        for bme in range(NBUF):
            s = NBUF * p + bme
            pre = (bme + NBUF - 1) % NBUF

            @pl.when(s >= 1)
            def _():
                writeback_wait(s - 1, pre)

            @pl.when(s + NBUF - 1 < S)
            def _():
                gather(s + NBUF - 1, pre)

            gather_wait(s, bme)
            writeback(s, bme)
        return 0

    lax.fori_loop(0, S // NBUF, quad, 0)
    writeback_wait(S - 1, (S - 1) % NBUF)


@jax.jit
def kernel(recipe_indices, embedding_weight):
    idx_t = jnp.transpose(recipe_indices.astype(jnp.int32))
    mesh = plsc.VectorSubcoreMesh(
        core_axis_name="c", subcore_axis_name="s",
        num_cores=NC, num_subcores=NS)
    return pl.kernel(
        _gather_body,
        out_type=jax.ShapeDtypeStruct((B, S, D), jnp.float32),
        mesh=mesh,
        scratch_types=[
            pltpu.VMEM((S, BPW), jnp.int32),
            pltpu.VMEM((NBUF, BPW, D), jnp.float32),
        ] + [pltpu.SemaphoreType.DMA] * 9,
        compiler_params=pltpu.CompilerParams(use_tc_tiling_on_sc=False),
    )(idx_t, embedding_weight)
